# Initial kernel scaffold; baseline (speedup 1.0000x reference)
import jax
import jax.numpy as jnp
from jax.experimental import pallas as pl


def kernel(x, params, edge_index):
    raise NotImplementedError("write your pallas kernel here")


# speedup vs baseline: 14.9264x; 14.9264x over previous
import numpy as np
import jax
import jax.numpy as jnp
from jax import lax
from jax.experimental import pallas as pl
from jax.experimental.pallas import tpu as pltpu
from jax.experimental.pallas import tpu_sc as plsc

N = 10000
NPAD = 10240
E = 160000
D = 128
H = 10
DH = 13
DHP = 16
FW = H * DHP
HP = 16

RB = 512
EB = 640
_GRID_N = NPAD // RB
_GRID_E = E // EB

EPT = E // 32
EPC = E // 2
CH = 128
NCH = EPT // CH
TL = EPT - NCH * CH
RPT = NPAD // 16

_f32 = jnp.float32


def _elu(x):
    return jnp.where(x > 0, x, jnp.exp(jnp.minimum(x, 0.0)) - 1.0)


def _lnorm(x, g, b):
    m = jnp.mean(x, axis=-1, keepdims=True)
    v = jnp.mean(jnp.square(x - m), axis=-1, keepdims=True)
    return (x - m) / jnp.sqrt(v + 1e-5) * g + b


_G_NP = np.zeros((FW, HP), np.float32)
for _h in range(H):
    _G_NP[_h * DHP:_h * DHP + DH, _h] = 1.0
_GE_NP = _G_NP / np.sqrt(float(DH))
_GX_NP = _G_NP.T.copy()


_SC_CACHE = {}


def _sc_mesh():
    if "mesh" not in _SC_CACHE:
        _SC_CACHE["mesh"] = plsc.VectorSubcoreMesh(
            core_axis_name="c", subcore_axis_name="s")
    return _SC_CACHE["mesh"]


def _sc_kernel(name, body, out_type, scratch_types):
    if name not in _SC_CACHE:
        _SC_CACHE[name] = pl.kernel(
            body, out_type=out_type, mesh=_sc_mesh(),
            scratch_types=scratch_types,
            compiler_params=pltpu.CompilerParams(use_tc_tiling_on_sc=False))
    return _SC_CACHE[name]


def _zero_rows(zb, wlanes):
    def body(i, _):
        for k in range(wlanes):
            zb[i, pl.ds(k * 16, 16)] = jnp.zeros((16,), _f32)
        return 0
    lax.fori_loop(0, zb.shape[0], body, 0)


def _deg_body(dst_hbm, out_hbm, dsti, dsti8, ones_v, zb, acc):
    c = lax.axis_index("c")
    s = lax.axis_index("s")

    def fill(i, _):
        ones_v[i, :] = jnp.ones((16,), _f32)
        zb[i, :] = jnp.zeros((16,), _f32)
        return 0
    lax.fori_loop(0, CH, fill, 0)
    for r in range(RPT // CH):
        pltpu.sync_copy(zb, acc.at[pl.ds(s * RPT + r * CH, CH)])
    plsc.subcore_barrier()

    eb = c * EPC + s * EPT

    def body(j, _):
        pltpu.sync_copy(dst_hbm.at[pl.ds(eb + j * CH, CH)], dsti)
        pltpu.sync_copy(ones_v, acc.at[dsti], add=True)
        return 0
    lax.fori_loop(0, NCH, body, 0)
    pltpu.sync_copy(dst_hbm.at[pl.ds(eb + NCH * CH, TL)], dsti8)
    pltpu.sync_copy(ones_v.at[pl.ds(0, TL)], acc.at[dsti8], add=True)

    plsc.subcore_barrier()
    pltpu.sync_copy(acc.at[pl.ds(s * RPT, RPT)],
                    out_hbm.at[pl.ds(c * NPAD + s * RPT, RPT)])


def _deg_call():
    return _sc_kernel(
        "deg",
        _deg_body,
        out_type=jax.ShapeDtypeStruct((2 * NPAD, HP), _f32),
        scratch_types=[
        pltpu.VMEM((CH,), jnp.int32),
        pltpu.VMEM((TL,), jnp.int32),
            pltpu.VMEM((CH, HP), _f32),
            pltpu.VMEM((CH, HP), _f32),
            pltpu.VMEM_SHARED((NPAD, HP), _f32),
        ],
    )


def _segsum_body(hn_hbm, src_hbm, dst_hbm, out_hbm,
                 srci, dsti, srci8, dsti8, rows, rows8, zb, acc, sem):
    c = lax.axis_index("c")
    s = lax.axis_index("s")

    _zero_rows(zb, D // 16)
    for r in range(RPT // CH):
        pltpu.sync_copy(zb, acc.at[pl.ds(s * RPT + r * CH, CH)])
    plsc.subcore_barrier()

    eb = c * EPC + s * EPT

    def body(j, _):
        pltpu.sync_copy(src_hbm.at[pl.ds(eb + j * CH, CH)], srci)
        pltpu.sync_copy(dst_hbm.at[pl.ds(eb + j * CH, CH)], dsti)
        pltpu.async_copy(hn_hbm.at[srci], rows, sem).wait()
        pltpu.sync_copy(rows, acc.at[dsti], add=True)
        return 0
    lax.fori_loop(0, NCH, body, 0)
    pltpu.sync_copy(src_hbm.at[pl.ds(eb + NCH * CH, TL)], srci8)
    pltpu.sync_copy(dst_hbm.at[pl.ds(eb + NCH * CH, TL)], dsti8)
    pltpu.async_copy(hn_hbm.at[srci8], rows8, sem).wait()
    pltpu.sync_copy(rows8, acc.at[dsti8], add=True)

    plsc.subcore_barrier()
    pltpu.sync_copy(acc.at[pl.ds(s * RPT, RPT)],
                    out_hbm.at[pl.ds(c * NPAD + s * RPT, RPT)])


def _segsum_call():
    return _sc_kernel(
        "segsum",
        _segsum_body,
        out_type=jax.ShapeDtypeStruct((2 * NPAD, D), _f32),
        scratch_types=[
        pltpu.VMEM((CH,), jnp.int32),
        pltpu.VMEM((CH,), jnp.int32),
        pltpu.VMEM((TL,), jnp.int32),
        pltpu.VMEM((TL,), jnp.int32),
        pltpu.VMEM((CH, D), _f32),
        pltpu.VMEM((TL, D), _f32),
        pltpu.VMEM((CH, D), _f32),
            pltpu.VMEM_SHARED((NPAD, D), _f32),
            pltpu.SemaphoreType.DMA,
        ],
    )


def _gath_body(feat_hbm, src_hbm, dst_hbm, fs_hbm, fd_hbm,
               srci, dsti, srci8, dsti8, fsb, fdb, fsb8, fdb8, sem, sem2):
    c = lax.axis_index("c")
    s = lax.axis_index("s")
    eb = (c * 16 + s) * EPT

    def body(j, _):
        pltpu.sync_copy(src_hbm.at[pl.ds(eb + j * CH, CH)], srci)
        pltpu.sync_copy(dst_hbm.at[pl.ds(eb + j * CH, CH)], dsti)
        d1 = pltpu.async_copy(feat_hbm.at[srci], fsb, sem)
        d2 = pltpu.async_copy(feat_hbm.at[dsti], fdb, sem2)
        d1.wait()
        d2.wait()
        pltpu.sync_copy(fsb, fs_hbm.at[pl.ds(eb + j * CH, CH)])
        pltpu.sync_copy(fdb, fd_hbm.at[pl.ds(eb + j * CH, CH)])
        return 0
    lax.fori_loop(0, NCH, body, 0)
    pltpu.sync_copy(src_hbm.at[pl.ds(eb + NCH * CH, TL)], srci8)
    pltpu.sync_copy(dst_hbm.at[pl.ds(eb + NCH * CH, TL)], dsti8)
    d1 = pltpu.async_copy(feat_hbm.at[srci8], fsb8, sem)
    d2 = pltpu.async_copy(feat_hbm.at[dsti8], fdb8, sem2)
    d1.wait()
    d2.wait()
    pltpu.sync_copy(fsb8, fs_hbm.at[pl.ds(eb + NCH * CH, TL)])
    pltpu.sync_copy(fdb8, fd_hbm.at[pl.ds(eb + NCH * CH, TL)])


def _gath_call():
    return _sc_kernel(
        "gath",
        _gath_body,
        out_type=(jax.ShapeDtypeStruct((E, FW), _f32),
                  jax.ShapeDtypeStruct((E, FW), _f32)),
        scratch_types=[
        pltpu.VMEM((CH,), jnp.int32),
        pltpu.VMEM((CH,), jnp.int32),
        pltpu.VMEM((TL,), jnp.int32),
        pltpu.VMEM((TL,), jnp.int32),
        pltpu.VMEM((CH, FW), _f32),
        pltpu.VMEM((CH, FW), _f32),
        pltpu.VMEM((TL, FW), _f32),
        pltpu.VMEM((TL, FW), _f32),
            pltpu.SemaphoreType.DMA,
            pltpu.SemaphoreType.DMA,
        ],
    )


CHS = 32
NCHS = EPT // CHS
TLS = EPT - NCHS * CHS


def _attscat_body(m_hbm, a_hbm, dst_hbm, rs_hbm, dn_hbm,
                  dsti, dsti8, mb, ab, mb8, ab8, zbm, zba, accm, acca):
    c = lax.axis_index("c")
    s = lax.axis_index("s")

    _zero_rows(zbm, FW // 16)
    _zero_rows(zba, HP // 16)

    def zcp(r, _):
        pltpu.sync_copy(zbm, accm.at[pl.ds(s * RPT + r * CHS, CHS)])
        pltpu.sync_copy(zba, acca.at[pl.ds(s * RPT + r * CHS, CHS)])
        return 0
    lax.fori_loop(0, RPT // CHS, zcp, 0)
    plsc.subcore_barrier()

    eb = c * EPC + s * EPT

    def body(j, _):
        rb = eb + j * CHS
        pltpu.sync_copy(dst_hbm.at[pl.ds(rb, CHS)], dsti)
        pltpu.sync_copy(m_hbm.at[pl.ds(rb, CHS)], mb)
        pltpu.sync_copy(a_hbm.at[pl.ds(rb, CHS)], ab)
        pltpu.sync_copy(mb, accm.at[dsti], add=True)
        pltpu.sync_copy(ab, acca.at[dsti], add=True)
        return 0
    lax.fori_loop(0, NCHS, body, 0)
    rb = eb + NCHS * CHS
    pltpu.sync_copy(dst_hbm.at[pl.ds(rb, TLS)], dsti8)
    pltpu.sync_copy(m_hbm.at[pl.ds(rb, TLS)], mb8)
    pltpu.sync_copy(a_hbm.at[pl.ds(rb, TLS)], ab8)
    pltpu.sync_copy(mb8, accm.at[dsti8], add=True)
    pltpu.sync_copy(ab8, acca.at[dsti8], add=True)

    plsc.subcore_barrier()
    pltpu.sync_copy(accm.at[pl.ds(s * RPT, RPT)],
                    rs_hbm.at[pl.ds(c * NPAD + s * RPT, RPT)])
    pltpu.sync_copy(acca.at[pl.ds(s * RPT, RPT)],
                    dn_hbm.at[pl.ds(c * NPAD + s * RPT, RPT)])


def _attscat_call():
    return _sc_kernel(
        "attscat",
        _attscat_body,
        out_type=(jax.ShapeDtypeStruct((2 * NPAD, FW), _f32),
                  jax.ShapeDtypeStruct((2 * NPAD, HP), _f32)),
        scratch_types=[
            pltpu.VMEM((CHS,), jnp.int32),
            pltpu.VMEM((TLS,), jnp.int32),
            pltpu.VMEM((CHS, FW), _f32),
            pltpu.VMEM((CHS, HP), _f32),
            pltpu.VMEM((TLS, FW), _f32),
            pltpu.VMEM((TLS, HP), _f32),
            pltpu.VMEM((CHS, FW), _f32),
            pltpu.VMEM((CHS, HP), _f32),
            pltpu.VMEM_SHARED((NPAD, FW), _f32),
            pltpu.VMEM_SHARED((NPAD, HP), _f32),
        ],
    )


def _row_spec(w):
    return pl.BlockSpec((RB, w), lambda i: (i, 0))


def _full_spec(shape):
    nd = len(shape)
    return pl.BlockSpec(shape, lambda i: (0,) * nd)


def _dot(a, b):
    return jnp.dot(a, b, preferred_element_type=_f32)


def _in_body(x_ref, wi, bi, g0, b0, hn_ref):
    h = _dot(x_ref[...], wi[...]) + bi[...]
    hn_ref[...] = _lnorm(h, g0[...], b0[...])


_in_call = pl.pallas_call(
    _in_body,
    grid=(_GRID_N,),
    in_specs=[_row_spec(D), _full_spec((D, D)), _full_spec((1, D)),
              _full_spec((1, D)), _full_spec((1, D))],
    out_specs=_row_spec(D),
    out_shape=jax.ShapeDtypeStruct((NPAD, D), _f32),
)


def _make_conv_fin(has_feat):
    def body(hn_ref, sp_ref, degp_ref, wself, wneigh, bias, ig, ib, siw, sib,
             ng, nb, *rest):
        if has_feat:
            watt = rest[0]
            outs = rest[1:]
        else:
            outs = rest
        hn = hn_ref[...]
        spv = sp_ref[...]
        sv = spv[0] + spv[1]
        dgv = degp_ref[...]
        deg = dgv[0, :, 0:1] + dgv[1, :, 0:1]
        mean = sv / jnp.maximum(deg, 1.0)
        conv = _dot(hn, wself[...]) + _dot(mean, wneigh[...]) + bias[...]
        h2 = _lnorm(conv + hn, ig[...], ib[...])
        hnew = h2 + _elu(_dot(h2, siw[...]) + sib[...])
        hn2 = _lnorm(hnew, ng[...], nb[...])
        outs[0][...] = hn2
        if has_feat:
            outs[1][...] = _dot(hn2, watt[...])
    return body


def _conv_fin_call(has_feat):
    in_specs = [
        _row_spec(D),
        pl.BlockSpec((2, RB, D), lambda i: (0, i, 0)),
        pl.BlockSpec((2, RB, HP), lambda i: (0, i, 0)),
        _full_spec((D, D)), _full_spec((D, D)), _full_spec((1, D)),
        _full_spec((1, D)), _full_spec((1, D)),
        _full_spec((D, D)), _full_spec((1, D)),
        _full_spec((1, D)), _full_spec((1, D)),
    ]
    out_specs = [_row_spec(D)]
    out_shape = [jax.ShapeDtypeStruct((NPAD, D), _f32)]
    if has_feat:
        in_specs.append(_full_spec((D, FW)))
        out_specs.append(_row_spec(FW))
        out_shape.append(jax.ShapeDtypeStruct((NPAD, FW), _f32))
    return pl.pallas_call(
        _make_conv_fin(has_feat),
        grid=(_GRID_N,),
        in_specs=in_specs,
        out_specs=out_specs,
        out_shape=out_shape,
    )


def _edge_body(fs_ref, fd_ref, ge, gx, m_ref, a_ref):
    fs = fs_ref[...]
    prod = fs * fd_ref[...]
    a = jnp.exp(_dot(prod, ge[...]))
    a_ref[...] = a
    m_ref[...] = _dot(a, gx[...]) * fs


_edge_call = pl.pallas_call(
    _edge_body,
    grid=(_GRID_E,),
    in_specs=[pl.BlockSpec((EB, FW), lambda i: (i, 0)),
              pl.BlockSpec((EB, FW), lambda i: (i, 0)),
              _full_spec((FW, HP)), _full_spec((HP, FW))],
    out_specs=[pl.BlockSpec((EB, FW), lambda i: (i, 0)),
               pl.BlockSpec((EB, HP), lambda i: (i, 0))],
    out_shape=[jax.ShapeDtypeStruct((E, FW), _f32),
               jax.ShapeDtypeStruct((E, HP), _f32)],
)


def _make_att_fin(is_final):
    def body(hn_ref, rs_ref, dn_ref, gx, hrw, hrb, ig, ib, f1w, f1b, f2w, f2b,
             *rest):
        hn = hn_ref[...]
        rsv = rs_ref[...]
        rs = rsv[0] + rsv[1]
        dnv = dn_ref[...]
        dn = dnv[0] + dnv[1]
        dexp = _dot(dn, gx[...])
        rst = rs / jnp.maximum(dexp, 1e-30)
        ho = _dot(_elu(rst), hrw[...]) + hrb[...]
        h2 = _lnorm(ho + hn, ig[...], ib[...])
        ff = _elu(_dot(_elu(_dot(h2, f1w[...]) + f1b[...]), f2w[...])
                  + f2b[...])
        hnew = h2 + ff
        if is_final:
            wout, bout, out_ref = rest
            out_ref[...] = _dot(hnew, wout[...]) + bout[...]
        else:
            ng, nb, watt, hn_out, feat_out = rest
            hn2 = _lnorm(hnew, ng[...], nb[...])
            hn_out[...] = hn2
            feat_out[...] = _dot(hn2, watt[...])
    return body


def _att_fin_call(is_final):
    in_specs = [
        _row_spec(D),
        pl.BlockSpec((2, RB, FW), lambda i: (0, i, 0)),
        pl.BlockSpec((2, RB, HP), lambda i: (0, i, 0)),
        _full_spec((HP, FW)),
        _full_spec((FW, D)), _full_spec((1, D)),
        _full_spec((1, D)), _full_spec((1, D)),
        _full_spec((D, 4 * D)), _full_spec((1, 4 * D)),
        _full_spec((4 * D, D)), _full_spec((1, D)),
    ]
    if is_final:
        in_specs += [_full_spec((D, D)), _full_spec((1, D))]
        out_specs = _row_spec(D)
        out_shape = jax.ShapeDtypeStruct((NPAD, D), _f32)
    else:
        in_specs += [_full_spec((1, D)), _full_spec((1, D)),
                     _full_spec((D, FW))]
        out_specs = [_row_spec(D), _row_spec(FW)]
        out_shape = [jax.ShapeDtypeStruct((NPAD, D), _f32),
                     jax.ShapeDtypeStruct((NPAD, FW), _f32)]
    return pl.pallas_call(
        _make_att_fin(is_final),
        grid=(_GRID_N,),
        in_specs=in_specs,
        out_specs=out_specs,
        out_shape=out_shape,
    )


_convfin_plain = _conv_fin_call(False)
_convfin_feat = _conv_fin_call(True)
_attfin_mid = _att_fin_call(False)
_attfin_last = _att_fin_call(True)


def _watt_pad(p):
    w = p["W_att"].reshape(D, H, DH)
    return jnp.pad(w, ((0, 0), (0, 0), (0, DHP - DH))).reshape(D, FW)


def _hrw_pad(p):
    w = p["hr_W"].reshape(H, DH, D)
    return jnp.pad(w, ((0, 0), (0, DHP - DH), (0, 0))).reshape(FW, D)


def kernel(x, params, edge_index):
    src = edge_index[0]
    dst = edge_index[1]
    xp = jnp.pad(x, ((0, NPAD - N), (0, 0)))
    r1 = lambda v: v.reshape(1, -1)
    ge = jnp.asarray(_GE_NP)
    gx = jnp.asarray(_GX_NP)

    degp = _deg_call()(dst).reshape(2, NPAD, HP)

    p0 = params["conv0"]
    hn = _in_call(xp, params["W_in"], r1(params["b_in"]),
                  r1(p0["ln_g"]), r1(p0["ln_b"]))

    feat = None
    for i in range(3):
        p = params["conv%d" % i]
        sp = _segsum_call()(hn, src, dst).reshape(2, NPAD, D)
        common = (hn, sp, degp, p["Wself"], p["Wneigh"], r1(p["bias"]),
                  r1(p["iln_g"]), r1(p["iln_b"]), p["si_W"], r1(p["si_b"]))
        if i < 2:
            q = params["conv%d" % (i + 1)]
            (hn,) = _convfin_plain(*common, r1(q["ln_g"]), r1(q["ln_b"]))
        else:
            q = params["att0"]
            hn, feat = _convfin_feat(*common, r1(q["ln_g"]), r1(q["ln_b"]),
                                     _watt_pad(q))

    for j in range(3):
        p = params["att%d" % j]
        fs, fd = _gath_call()(feat, src, dst)
        m, a = _edge_call(fs, fd, ge, gx)
        rsp, dnp = _attscat_call()(m, a, dst)
        common = (hn, rsp.reshape(2, NPAD, FW), dnp.reshape(2, NPAD, HP),
                  gx, _hrw_pad(p), r1(p["hr_b"]),
                  r1(p["iln_g"]), r1(p["iln_b"]),
                  p["ff1_W"], r1(p["ff1_b"]), p["ff2_W"], r1(p["ff2_b"]))
        if j < 2:
            q = params["att%d" % (j + 1)]
            hn, feat = _attfin_mid(*common, r1(q["ln_g"]), r1(q["ln_b"]),
                                   _watt_pad(q))
        else:
            out = _attfin_last(*common, params["W_out"], r1(params["b_out"]))

    return out[:N]


# speedup vs baseline: 18.8298x; 1.2615x over previous
import numpy as np
import jax
import jax.numpy as jnp
from jax import lax
from jax.experimental import pallas as pl
from jax.experimental.pallas import tpu as pltpu
from jax.experimental.pallas import tpu_sc as plsc

N = 10000
NPAD = 10240
E = 160000
D = 128
H = 10
DH = 13
DHP = 16
FW = H * DHP
HP = 16

RB = 512
EB = 640
_GRID_N = NPAD // RB
_GRID_E = E // EB

EPT = E // 32
EPC = E // 2
CH = 128
NCH = EPT // CH
TL = EPT - NCH * CH
RPT = NPAD // 16

_f32 = jnp.float32


def _elu(x):
    return jnp.where(x > 0, x, jnp.exp(jnp.minimum(x, 0.0)) - 1.0)


def _lnorm(x, g, b):
    m = jnp.mean(x, axis=-1, keepdims=True)
    v = jnp.mean(jnp.square(x - m), axis=-1, keepdims=True)
    return (x - m) / jnp.sqrt(v + 1e-5) * g + b


_G_NP = np.zeros((FW, HP), np.float32)
for _h in range(H):
    _G_NP[_h * DHP:_h * DHP + DH, _h] = 1.0
_GE_NP = _G_NP / np.sqrt(float(DH))
_GX_NP = _G_NP.T.copy()


_SC_CACHE = {}


def _sc_mesh():
    if "mesh" not in _SC_CACHE:
        _SC_CACHE["mesh"] = plsc.VectorSubcoreMesh(
            core_axis_name="c", subcore_axis_name="s")
    return _SC_CACHE["mesh"]


def _sc_kernel(name, body, out_type, scratch_types):
    if name not in _SC_CACHE:
        _SC_CACHE[name] = pl.kernel(
            body, out_type=out_type, mesh=_sc_mesh(),
            scratch_types=scratch_types,
            compiler_params=pltpu.CompilerParams(use_tc_tiling_on_sc=False))
    return _SC_CACHE[name]


def _zero_rows(zb, wlanes):
    def body(i, _):
        for k in range(wlanes):
            zb[i, pl.ds(k * 16, 16)] = jnp.zeros((16,), _f32)
        return 0
    lax.fori_loop(0, zb.shape[0], body, 0)


def _deg_body(dst_hbm, out_hbm, dsti, dsti8, ones_v, zb, acc):
    c = lax.axis_index("c")
    s = lax.axis_index("s")

    def fill(i, _):
        ones_v[i, :] = jnp.ones((16,), _f32)
        zb[i, :] = jnp.zeros((16,), _f32)
        return 0
    lax.fori_loop(0, CH, fill, 0)
    for r in range(RPT // CH):
        pltpu.sync_copy(zb, acc.at[pl.ds(s * RPT + r * CH, CH)])
    plsc.subcore_barrier()

    eb = c * EPC + s * EPT

    def body(j, _):
        pltpu.sync_copy(dst_hbm.at[pl.ds(eb + j * CH, CH)], dsti)
        pltpu.sync_copy(ones_v, acc.at[dsti], add=True)
        return 0
    lax.fori_loop(0, NCH, body, 0)
    pltpu.sync_copy(dst_hbm.at[pl.ds(eb + NCH * CH, TL)], dsti8)
    pltpu.sync_copy(ones_v.at[pl.ds(0, TL)], acc.at[dsti8], add=True)

    plsc.subcore_barrier()
    pltpu.sync_copy(acc.at[pl.ds(s * RPT, RPT)],
                    out_hbm.at[pl.ds(c * NPAD + s * RPT, RPT)])


def _deg_call():
    return _sc_kernel(
        "deg",
        _deg_body,
        out_type=jax.ShapeDtypeStruct((2 * NPAD, HP), _f32),
        scratch_types=[
        pltpu.VMEM((CH,), jnp.int32),
        pltpu.VMEM((TL,), jnp.int32),
            pltpu.VMEM((CH, HP), _f32),
            pltpu.VMEM((CH, HP), _f32),
            pltpu.VMEM_SHARED((NPAD, HP), _f32),
        ],
    )


def _segsum_body(hn_hbm, src_hbm, dst_hbm, out_hbm,
                 srci, dsti, srci1, dsti1, srci8, dsti8,
                 rows, rows1, rows8, zb, acc, sem, sem1):
    c = lax.axis_index("c")
    s = lax.axis_index("s")

    _zero_rows(zb, D // 16)
    for r in range(RPT // 64):
        pltpu.sync_copy(zb, acc.at[pl.ds(s * RPT + r * 64, 64)])
    plsc.subcore_barrier()

    eb = c * EPC + s * EPT

    def ld(j, sref, dref):
        pltpu.sync_copy(src_hbm.at[pl.ds(eb + j * CH, CH)], sref)
        pltpu.sync_copy(dst_hbm.at[pl.ds(eb + j * CH, CH)], dref)

    ld(0, srci, dsti)
    pltpu.async_copy(hn_hbm.at[srci], rows, sem)

    def body(k, _):
        j = 2 * k
        ld(j + 1, srci1, dsti1)
        pltpu.async_copy(hn_hbm.at[srci1], rows1, sem1)
        pltpu.make_async_copy(hn_hbm.at[srci], rows, sem).wait()
        pltpu.sync_copy(rows, acc.at[dsti], add=True)
        ld(j + 2, srci, dsti)
        pltpu.async_copy(hn_hbm.at[srci], rows, sem)
        pltpu.make_async_copy(hn_hbm.at[srci1], rows1, sem1).wait()
        pltpu.sync_copy(rows1, acc.at[dsti1], add=True)
        return 0
    lax.fori_loop(0, (NCH - 1) // 2, body, 0)
    pltpu.make_async_copy(hn_hbm.at[srci], rows, sem).wait()
    pltpu.sync_copy(rows, acc.at[dsti], add=True)

    pltpu.sync_copy(src_hbm.at[pl.ds(eb + NCH * CH, TL)], srci8)
    pltpu.sync_copy(dst_hbm.at[pl.ds(eb + NCH * CH, TL)], dsti8)
    pltpu.async_copy(hn_hbm.at[srci8], rows8, sem).wait()
    pltpu.sync_copy(rows8, acc.at[dsti8], add=True)

    plsc.subcore_barrier()
    pltpu.sync_copy(acc.at[pl.ds(s * RPT, RPT)],
                    out_hbm.at[pl.ds(c * NPAD + s * RPT, RPT)])


def _segsum_call():
    return _sc_kernel(
        "segsum",
        _segsum_body,
        out_type=jax.ShapeDtypeStruct((2 * NPAD, D), _f32),
        scratch_types=[
            pltpu.VMEM((CH,), jnp.int32),
            pltpu.VMEM((CH,), jnp.int32),
            pltpu.VMEM((CH,), jnp.int32),
            pltpu.VMEM((CH,), jnp.int32),
            pltpu.VMEM((TL,), jnp.int32),
            pltpu.VMEM((TL,), jnp.int32),
            pltpu.VMEM((CH, D), _f32),
            pltpu.VMEM((CH, D), _f32),
            pltpu.VMEM((TL, D), _f32),
            pltpu.VMEM((64, D), _f32),
            pltpu.VMEM_SHARED((NPAD, D), _f32),
            pltpu.SemaphoreType.DMA,
            pltpu.SemaphoreType.DMA,
        ],
    )


def _gath_body(feat_hbm, src_hbm, dst_hbm, fs_hbm, fd_hbm,
               srci, dsti, srci1, dsti1, srci8, dsti8,
               fsb, fdb, fsb1, fdb1, fsb8, fdb8, sem, sem2, sem3, sem4):
    c = lax.axis_index("c")
    s = lax.axis_index("s")
    eb = (c * 16 + s) * EPT

    def ld(j, sref, dref):
        pltpu.sync_copy(src_hbm.at[pl.ds(eb + j * CH, CH)], sref)
        pltpu.sync_copy(dst_hbm.at[pl.ds(eb + j * CH, CH)], dref)

    ld(0, srci, dsti)
    pltpu.async_copy(feat_hbm.at[srci], fsb, sem)
    pltpu.async_copy(feat_hbm.at[dsti], fdb, sem2)

    def body(k, _):
        j = 2 * k
        ld(j + 1, srci1, dsti1)
        pltpu.async_copy(feat_hbm.at[srci1], fsb1, sem3)
        pltpu.async_copy(feat_hbm.at[dsti1], fdb1, sem4)
        pltpu.make_async_copy(feat_hbm.at[srci], fsb, sem).wait()
        pltpu.make_async_copy(feat_hbm.at[dsti], fdb, sem2).wait()
        pltpu.sync_copy(fsb, fs_hbm.at[pl.ds(eb + j * CH, CH)])
        pltpu.sync_copy(fdb, fd_hbm.at[pl.ds(eb + j * CH, CH)])
        ld(j + 2, srci, dsti)
        pltpu.async_copy(feat_hbm.at[srci], fsb, sem)
        pltpu.async_copy(feat_hbm.at[dsti], fdb, sem2)
        pltpu.make_async_copy(feat_hbm.at[srci1], fsb1, sem3).wait()
        pltpu.make_async_copy(feat_hbm.at[dsti1], fdb1, sem4).wait()
        pltpu.sync_copy(fsb1, fs_hbm.at[pl.ds(eb + (j + 1) * CH, CH)])
        pltpu.sync_copy(fdb1, fd_hbm.at[pl.ds(eb + (j + 1) * CH, CH)])
        return 0
    lax.fori_loop(0, (NCH - 1) // 2, body, 0)
    pltpu.make_async_copy(feat_hbm.at[srci], fsb, sem).wait()
    pltpu.make_async_copy(feat_hbm.at[dsti], fdb, sem2).wait()
    pltpu.sync_copy(fsb, fs_hbm.at[pl.ds(eb + (NCH - 1) * CH, CH)])
    pltpu.sync_copy(fdb, fd_hbm.at[pl.ds(eb + (NCH - 1) * CH, CH)])

    pltpu.sync_copy(src_hbm.at[pl.ds(eb + NCH * CH, TL)], srci8)
    pltpu.sync_copy(dst_hbm.at[pl.ds(eb + NCH * CH, TL)], dsti8)
    d1 = pltpu.async_copy(feat_hbm.at[srci8], fsb8, sem)
    d2 = pltpu.async_copy(feat_hbm.at[dsti8], fdb8, sem2)
    d1.wait()
    d2.wait()
    pltpu.sync_copy(fsb8, fs_hbm.at[pl.ds(eb + NCH * CH, TL)])
    pltpu.sync_copy(fdb8, fd_hbm.at[pl.ds(eb + NCH * CH, TL)])


def _gath_call():
    return _sc_kernel(
        "gath",
        _gath_body,
        out_type=(jax.ShapeDtypeStruct((E, FW), _f32),
                  jax.ShapeDtypeStruct((E, FW), _f32)),
        scratch_types=[
            pltpu.VMEM((CH,), jnp.int32),
            pltpu.VMEM((CH,), jnp.int32),
            pltpu.VMEM((CH,), jnp.int32),
            pltpu.VMEM((CH,), jnp.int32),
            pltpu.VMEM((TL,), jnp.int32),
            pltpu.VMEM((TL,), jnp.int32),
            pltpu.VMEM((CH, FW), _f32),
            pltpu.VMEM((CH, FW), _f32),
            pltpu.VMEM((CH, FW), _f32),
            pltpu.VMEM((CH, FW), _f32),
            pltpu.VMEM((TL, FW), _f32),
            pltpu.VMEM((TL, FW), _f32),
            pltpu.SemaphoreType.DMA,
            pltpu.SemaphoreType.DMA,
            pltpu.SemaphoreType.DMA,
            pltpu.SemaphoreType.DMA,
        ],
    )


CHS = 40
NCHS = EPT // CHS


def _attscat_body(m_hbm, a_hbm, dst_hbm, z_hbm, rs_hbm, dn_hbm,
                  dsti, dsti1, mb, mb1, ab, ab1, zba, accm, acca,
                  sd0, sm0, sa0, sd1, sm1, sa1):
    c = lax.axis_index("c")
    s = lax.axis_index("s")

    pltpu.sync_copy(z_hbm.at[pl.ds(s * RPT, RPT)],
                    accm.at[pl.ds(s * RPT, RPT)])
    _zero_rows(zba, HP // 16)

    def zcp(r, _):
        pltpu.sync_copy(zba, acca.at[pl.ds(s * RPT + r * CHS, CHS)])
        return 0
    lax.fori_loop(0, RPT // CHS, zcp, 0)
    plsc.subcore_barrier()

    eb = c * EPC + s * EPT

    def lda(j, dref, mref, aref, s1, s2, s3):
        rb = eb + j * CHS
        pltpu.async_copy(dst_hbm.at[pl.ds(rb, CHS)], dref, s1)
        pltpu.async_copy(m_hbm.at[pl.ds(rb, CHS)], mref, s2)
        pltpu.async_copy(a_hbm.at[pl.ds(rb, CHS)], aref, s3)

    def wta(j, dref, mref, aref, s1, s2, s3):
        rb = eb + j * CHS
        pltpu.make_async_copy(dst_hbm.at[pl.ds(rb, CHS)], dref, s1).wait()
        pltpu.make_async_copy(m_hbm.at[pl.ds(rb, CHS)], mref, s2).wait()
        pltpu.make_async_copy(a_hbm.at[pl.ds(rb, CHS)], aref, s3).wait()

    lda(0, dsti, mb, ab, sd0, sm0, sa0)

    def body(k, _):
        j = 2 * k
        lda(j + 1, dsti1, mb1, ab1, sd1, sm1, sa1)
        wta(j, dsti, mb, ab, sd0, sm0, sa0)
        pltpu.sync_copy(mb, accm.at[dsti], add=True)
        pltpu.sync_copy(ab, acca.at[dsti], add=True)
        lda(j + 2, dsti, mb, ab, sd0, sm0, sa0)
        wta(j + 1, dsti1, mb1, ab1, sd1, sm1, sa1)
        pltpu.sync_copy(mb1, accm.at[dsti1], add=True)
        pltpu.sync_copy(ab1, acca.at[dsti1], add=True)
        return 0
    lax.fori_loop(0, (NCHS - 1) // 2, body, 0)
    wta(NCHS - 1, dsti, mb, ab, sd0, sm0, sa0)
    pltpu.sync_copy(mb, accm.at[dsti], add=True)
    pltpu.sync_copy(ab, acca.at[dsti], add=True)

    plsc.subcore_barrier()
    pltpu.sync_copy(accm.at[pl.ds(s * RPT, RPT)],
                    rs_hbm.at[pl.ds(c * NPAD + s * RPT, RPT)])
    pltpu.sync_copy(acca.at[pl.ds(s * RPT, RPT)],
                    dn_hbm.at[pl.ds(c * NPAD + s * RPT, RPT)])


def _attscat_call():
    return _sc_kernel(
        "attscat",
        _attscat_body,
        out_type=(jax.ShapeDtypeStruct((2 * NPAD, FW), _f32),
                  jax.ShapeDtypeStruct((2 * NPAD, HP), _f32)),
        scratch_types=[
            pltpu.VMEM((CHS,), jnp.int32),
            pltpu.VMEM((CHS,), jnp.int32),
            pltpu.VMEM((CHS, FW), _f32),
            pltpu.VMEM((CHS, FW), _f32),
            pltpu.VMEM((CHS, HP), _f32),
            pltpu.VMEM((CHS, HP), _f32),
            pltpu.VMEM((CHS, HP), _f32),
            pltpu.VMEM_SHARED((NPAD, FW), _f32),
            pltpu.VMEM_SHARED((NPAD, HP), _f32),
            pltpu.SemaphoreType.DMA,
            pltpu.SemaphoreType.DMA,
            pltpu.SemaphoreType.DMA,
            pltpu.SemaphoreType.DMA,
            pltpu.SemaphoreType.DMA,
            pltpu.SemaphoreType.DMA,
        ],
    )


def _row_spec(w):
    return pl.BlockSpec((RB, w), lambda i: (i, 0))


def _full_spec(shape):
    nd = len(shape)
    return pl.BlockSpec(shape, lambda i: (0,) * nd)


def _dot(a, b):
    return jnp.dot(a, b, preferred_element_type=_f32)


def _in_body(x_ref, wi, bi, g0, b0, hn_ref):
    h = _dot(x_ref[...], wi[...]) + bi[...]
    hn_ref[...] = _lnorm(h, g0[...], b0[...])


_in_call = pl.pallas_call(
    _in_body,
    grid=(_GRID_N,),
    in_specs=[_row_spec(D), _full_spec((D, D)), _full_spec((1, D)),
              _full_spec((1, D)), _full_spec((1, D))],
    out_specs=_row_spec(D),
    out_shape=jax.ShapeDtypeStruct((NPAD, D), _f32),
)


def _make_conv_fin(has_feat):
    def body(hn_ref, sp_ref, degp_ref, wself, wneigh, bias, ig, ib, siw, sib,
             ng, nb, *rest):
        if has_feat:
            watt = rest[0]
            outs = rest[1:]
        else:
            outs = rest
        hn = hn_ref[...]
        spv = sp_ref[...]
        sv = spv[0] + spv[1]
        dgv = degp_ref[...]
        deg = dgv[0, :, 0:1] + dgv[1, :, 0:1]
        mean = sv / jnp.maximum(deg, 1.0)
        conv = _dot(hn, wself[...]) + _dot(mean, wneigh[...]) + bias[...]
        h2 = _lnorm(conv + hn, ig[...], ib[...])
        hnew = h2 + _elu(_dot(h2, siw[...]) + sib[...])
        hn2 = _lnorm(hnew, ng[...], nb[...])
        outs[0][...] = hn2
        if has_feat:
            outs[1][...] = _dot(hn2, watt[...])
    return body


def _conv_fin_call(has_feat):
    in_specs = [
        _row_spec(D),
        pl.BlockSpec((2, RB, D), lambda i: (0, i, 0)),
        pl.BlockSpec((2, RB, HP), lambda i: (0, i, 0)),
        _full_spec((D, D)), _full_spec((D, D)), _full_spec((1, D)),
        _full_spec((1, D)), _full_spec((1, D)),
        _full_spec((D, D)), _full_spec((1, D)),
        _full_spec((1, D)), _full_spec((1, D)),
    ]
    out_specs = [_row_spec(D)]
    out_shape = [jax.ShapeDtypeStruct((NPAD, D), _f32)]
    if has_feat:
        in_specs.append(_full_spec((D, FW)))
        out_specs.append(_row_spec(FW))
        out_shape.append(jax.ShapeDtypeStruct((NPAD, FW), _f32))
    return pl.pallas_call(
        _make_conv_fin(has_feat),
        grid=(_GRID_N,),
        in_specs=in_specs,
        out_specs=out_specs,
        out_shape=out_shape,
    )


def _edge_body(fs_ref, fd_ref, ge, gx, m_ref, a_ref):
    fs = fs_ref[...]
    prod = fs * fd_ref[...]
    a = jnp.exp(_dot(prod, ge[...]))
    a_ref[...] = a
    m_ref[...] = _dot(a, gx[...]) * fs


_edge_call = pl.pallas_call(
    _edge_body,
    grid=(_GRID_E,),
    in_specs=[pl.BlockSpec((EB, FW), lambda i: (i, 0)),
              pl.BlockSpec((EB, FW), lambda i: (i, 0)),
              _full_spec((FW, HP)), _full_spec((HP, FW))],
    out_specs=[pl.BlockSpec((EB, FW), lambda i: (i, 0)),
               pl.BlockSpec((EB, HP), lambda i: (i, 0))],
    out_shape=[jax.ShapeDtypeStruct((E, FW), _f32),
               jax.ShapeDtypeStruct((E, HP), _f32)],
)


def _make_att_fin(is_final):
    def body(hn_ref, rs_ref, dn_ref, gx, hrw, hrb, ig, ib, f1w, f1b, f2w, f2b,
             *rest):
        hn = hn_ref[...]
        rsv = rs_ref[...]
        rs = rsv[0] + rsv[1]
        dnv = dn_ref[...]
        dn = dnv[0] + dnv[1]
        dexp = _dot(dn, gx[...])
        rst = rs / jnp.maximum(dexp, 1e-30)
        ho = _dot(_elu(rst), hrw[...]) + hrb[...]
        h2 = _lnorm(ho + hn, ig[...], ib[...])
        ff = _elu(_dot(_elu(_dot(h2, f1w[...]) + f1b[...]), f2w[...])
                  + f2b[...])
        hnew = h2 + ff
        if is_final:
            wout, bout, out_ref = rest
            out_ref[...] = _dot(hnew, wout[...]) + bout[...]
        else:
            ng, nb, watt, hn_out, feat_out = rest
            hn2 = _lnorm(hnew, ng[...], nb[...])
            hn_out[...] = hn2
            feat_out[...] = _dot(hn2, watt[...])
    return body


def _att_fin_call(is_final):
    in_specs = [
        _row_spec(D),
        pl.BlockSpec((2, RB, FW), lambda i: (0, i, 0)),
        pl.BlockSpec((2, RB, HP), lambda i: (0, i, 0)),
        _full_spec((HP, FW)),
        _full_spec((FW, D)), _full_spec((1, D)),
        _full_spec((1, D)), _full_spec((1, D)),
        _full_spec((D, 4 * D)), _full_spec((1, 4 * D)),
        _full_spec((4 * D, D)), _full_spec((1, D)),
    ]
    if is_final:
        in_specs += [_full_spec((D, D)), _full_spec((1, D))]
        out_specs = _row_spec(D)
        out_shape = jax.ShapeDtypeStruct((NPAD, D), _f32)
    else:
        in_specs += [_full_spec((1, D)), _full_spec((1, D)),
                     _full_spec((D, FW))]
        out_specs = [_row_spec(D), _row_spec(FW)]
        out_shape = [jax.ShapeDtypeStruct((NPAD, D), _f32),
                     jax.ShapeDtypeStruct((NPAD, FW), _f32)]
    return pl.pallas_call(
        _make_att_fin(is_final),
        grid=(_GRID_N,),
        in_specs=in_specs,
        out_specs=out_specs,
        out_shape=out_shape,
    )


_convfin_plain = _conv_fin_call(False)
_convfin_feat = _conv_fin_call(True)
_attfin_mid = _att_fin_call(False)
_attfin_last = _att_fin_call(True)


def _watt_pad(p):
    w = p["W_att"].reshape(D, H, DH)
    return jnp.pad(w, ((0, 0), (0, 0), (0, DHP - DH))).reshape(D, FW)


def _hrw_pad(p):
    w = p["hr_W"].reshape(H, DH, D)
    return jnp.pad(w, ((0, 0), (0, DHP - DH), (0, 0))).reshape(FW, D)


def kernel(x, params, edge_index):
    src = edge_index[0]
    dst = edge_index[1]
    xp = jnp.pad(x, ((0, NPAD - N), (0, 0)))
    r1 = lambda v: v.reshape(1, -1)
    ge = jnp.asarray(_GE_NP)
    gx = jnp.asarray(_GX_NP)
    zfw = jnp.zeros((NPAD, FW), _f32)

    degp = _deg_call()(dst).reshape(2, NPAD, HP)

    p0 = params["conv0"]
    hn = _in_call(xp, params["W_in"], r1(params["b_in"]),
                  r1(p0["ln_g"]), r1(p0["ln_b"]))

    feat = None
    for i in range(3):
        p = params["conv%d" % i]
        sp = _segsum_call()(hn, src, dst).reshape(2, NPAD, D)
        common = (hn, sp, degp, p["Wself"], p["Wneigh"], r1(p["bias"]),
                  r1(p["iln_g"]), r1(p["iln_b"]), p["si_W"], r1(p["si_b"]))
        if i < 2:
            q = params["conv%d" % (i + 1)]
            (hn,) = _convfin_plain(*common, r1(q["ln_g"]), r1(q["ln_b"]))
        else:
            q = params["att0"]
            hn, feat = _convfin_feat(*common, r1(q["ln_g"]), r1(q["ln_b"]),
                                     _watt_pad(q))

    for j in range(3):
        p = params["att%d" % j]
        fs, fd = _gath_call()(feat, src, dst)
        m, a = _edge_call(fs, fd, ge, gx)
        rsp, dnp = _attscat_call()(m, a, dst, zfw)
        common = (hn, rsp.reshape(2, NPAD, FW), dnp.reshape(2, NPAD, HP),
                  gx, _hrw_pad(p), r1(p["hr_b"]),
                  r1(p["iln_g"]), r1(p["iln_b"]),
                  p["ff1_W"], r1(p["ff1_b"]), p["ff2_W"], r1(p["ff2_b"]))
        if j < 2:
            q = params["att%d" % (j + 1)]
            hn, feat = _attfin_mid(*common, r1(q["ln_g"]), r1(q["ln_b"]),
                                   _watt_pad(q))
        else:
            out = _attfin_last(*common, params["W_out"], r1(params["b_out"]))

    return out[:N]
